# baseline (speedup 1.0000x reference)
import jax
import jax.numpy as jnp
import numpy as np
from jax.experimental import pallas as pl

BN = 512
RBLK = 512
PBLK = 512
LBLK = 8


def _rank_kernel(s_ref, st_ref, rank_ref):
    i = pl.program_id(0)
    s_i = s_ref[...]
    s_j = st_ref[...]
    gt = (s_j > s_i).astype(jnp.float32)
    idx_i = i * RBLK + jax.lax.broadcasted_iota(jnp.int32, (RBLK, 1), 0)
    idx_j = jax.lax.broadcasted_iota(jnp.int32, (RBLK, s_j.shape[1]), 1)
    tie = ((s_j == s_i) & (idx_j < idx_i)).astype(jnp.float32)
    rank_ref[...] = jnp.sum(gt + tie, axis=1, keepdims=True)


def _pool_kernel(rt_ref, x_ref, s_ref, o_ref, ts_ref):
    p = pl.program_id(0)
    ranks = rt_ref[...]
    p_col = (p * PBLK + jax.lax.broadcasted_iota(jnp.int32, (PBLK, 1), 0)).astype(jnp.float32)
    P = (ranks == p_col).astype(jnp.float32)
    dot = lambda a, b: jax.lax.dot_general(a, b, (((1,), (0,)), ((), ())),
                                           preferred_element_type=jnp.float32,
                                           precision=jax.lax.Precision.HIGHEST)
    rows = dot(P, x_ref[...])
    ts = dot(P, s_ref[...])
    ts_ref[...] = ts
    o_ref[...] = rows * jnp.tanh(ts)


def _mlp_kernel(xt_ref, pt_ref, W1x_ref, b1x_ref, W2x_ref, b2x_ref, gx_ref, bx_ref,
                W1s_ref, b1s_ref, W2s_ref, b2s_ref, gs_ref, bs_ref, oa_ref, ob_ref):
    H = W1x_ref.shape[0]

    def rep(row, W1c, b1c, W2t, b2c, gc, bc):
        t1 = jnp.tanh(W1c * row + b1c)
        t2 = jax.lax.dot_general(W2t, t1.astype(jnp.bfloat16),
                                 (((1,), (0,)), ((), ())),
                                 preferred_element_type=jnp.float32) + b2c
        h = jax.nn.relu(t2)
        mu = jnp.mean(h, axis=0, keepdims=True)
        var = jnp.mean((h - mu) ** 2, axis=0, keepdims=True)
        ln = (h - mu) / jnp.sqrt(var + 1e-5) * gc + bc
        m = ln.shape[1] // H
        return ln.astype(jnp.bfloat16).reshape(H, m, H).transpose(1, 2, 0).reshape(m, H * H)

    for l in range(LBLK):
        a = rep(xt_ref[l:l + 1, :], W1x_ref[...], b1x_ref[...], W2x_ref[...],
                b2x_ref[...], gx_ref[...], bx_ref[...])
        oa_ref[pl.ds(l * a.shape[0], a.shape[0]), :] = a
        b = rep(pt_ref[l:l + 1, :], W1s_ref[...], b1s_ref[...], W2s_ref[...],
                b2s_ref[...], gs_ref[...], bs_ref[...])
        ob_ref[pl.ds(l * b.shape[0], b.shape[0]), :] = b


def _reprs(xt, pt, wx, ws):
    L, N = xt.shape
    kk = pt.shape[1]
    H = wx[0].shape[0]
    wcol = pl.BlockSpec((H, 1), lambda i: (0, 0))
    wmat = pl.BlockSpec((H, H), lambda i: (0, 0))
    wspecs = [wcol, wcol, wmat, wcol, wcol, wcol]
    return pl.pallas_call(
        _mlp_kernel,
        grid=(L // LBLK,),
        in_specs=[pl.BlockSpec((LBLK, N), lambda i: (i, 0)),
                  pl.BlockSpec((LBLK, kk), lambda i: (i, 0))] + wspecs + wspecs,
        out_specs=(pl.BlockSpec((LBLK * N // H, N), lambda i: (i, 0)),
                   pl.BlockSpec((LBLK * kk // H, N), lambda i: (i, 0))),
        out_shape=(jax.ShapeDtypeStruct((L * N // H, N), jnp.bfloat16),
                   jax.ShapeDtypeStruct((L * kk // H, N), jnp.bfloat16)),
    )(xt, pt, *wx, *ws)


def _sim_kernel(a_ref, b_ref, o_ref):
    s = jax.lax.dot_general(a_ref[...], b_ref[...], (((1,), (0,)), ((), ())),
                            preferred_element_type=jnp.float32)
    m = jnp.max(s, axis=-1, keepdims=True)
    e = jnp.exp(s - m)
    o_ref[...] = e / jnp.sum(e, axis=-1, keepdims=True)


def kernel(x, edge_index, w_pool, xW1, xb1, xW2, xb2, xg, xbeta, sW1, sb1, sW2, sb2, sg, sbeta):
    N, L = x.shape
    H = xW1.shape[1]
    k = int(np.ceil(0.5 * N))

    score = jnp.matmul(x.astype(jnp.bfloat16), w_pool.astype(jnp.bfloat16),
                       preferred_element_type=jnp.float32) / jnp.linalg.norm(w_pool)
    s2d = score.reshape(N, 1)
    st = score.reshape(1, N)

    rank = pl.pallas_call(
        _rank_kernel,
        grid=(N // RBLK,),
        in_specs=[
            pl.BlockSpec((RBLK, 1), lambda i: (i, 0)),
            pl.BlockSpec((1, N), lambda i: (0, 0)),
        ],
        out_specs=pl.BlockSpec((RBLK, 1), lambda i: (i, 0)),
        out_shape=jax.ShapeDtypeStruct((N, 1), jnp.float32),
    )(s2d, st)

    pooled_x, _ts = pl.pallas_call(
        _pool_kernel,
        grid=(k // PBLK,),
        in_specs=[pl.BlockSpec((1, N), lambda p: (0, 0)),
                  pl.BlockSpec((N, L), lambda p: (0, 0)),
                  pl.BlockSpec((N, 1), lambda p: (0, 0))],
        out_specs=(pl.BlockSpec((PBLK, L), lambda p: (p, 0)),
                   pl.BlockSpec((PBLK, 1), lambda p: (p, 0))),
        out_shape=(jax.ShapeDtypeStruct((k, L), jnp.float32),
                   jax.ShapeDtypeStruct((k, 1), jnp.float32)),
    )(rank.reshape(1, N), x, s2d)

    xt = jnp.transpose(x)
    pt = jnp.transpose(pooled_x)
    prep = lambda W1, b1, W2, b2, g, beta: (
        W1.reshape(H, 1), b1.reshape(H, 1),
        jnp.transpose(W2.astype(jnp.bfloat16)), b2.reshape(H, 1),
        g.reshape(H, 1), beta.reshape(H, 1))
    node_flat, sup_flat = _reprs(xt, pt,
                                 prep(xW1, xb1, xW2, xb2, xg, xbeta),
                                 prep(sW1, sb1, sW2, sb2, sg, sbeta))

    sup_t = jnp.transpose(sup_flat)
    out = pl.pallas_call(
        _sim_kernel,
        grid=(N // BN,),
        in_specs=[
            pl.BlockSpec((BN, L * H), lambda i: (i, 0)),
            pl.BlockSpec((L * H, k), lambda i: (0, 0)),
        ],
        out_specs=pl.BlockSpec((BN, k), lambda i: (i, 0)),
        out_shape=jax.ShapeDtypeStruct((N, k), jnp.float32),
    )(node_flat, sup_t)
    return out


# speedup vs baseline: 1.0517x; 1.0517x over previous
import jax
import jax.numpy as jnp
import numpy as np
from jax.experimental import pallas as pl

BN = 512
RBLK = 512
PBLK = 512
LBLK = 8


def _rank_kernel(s_ref, st_ref, rank_ref):
    i = pl.program_id(0)
    s_i = s_ref[...]
    s_j = st_ref[...]
    gt = (s_j > s_i).astype(jnp.float32)
    idx_i = i * RBLK + jax.lax.broadcasted_iota(jnp.int32, (RBLK, 1), 0)
    idx_j = jax.lax.broadcasted_iota(jnp.int32, (RBLK, s_j.shape[1]), 1)
    tie = ((s_j == s_i) & (idx_j < idx_i)).astype(jnp.float32)
    rank_ref[...] = jnp.sum(gt + tie, axis=1, keepdims=True)


def _pool_kernel(rt_ref, x_ref, s_ref, o_ref, ts_ref):
    p = pl.program_id(0)
    ranks = rt_ref[...]
    p_col = (p * PBLK + jax.lax.broadcasted_iota(jnp.int32, (PBLK, 1), 0)).astype(jnp.float32)
    P = (ranks == p_col).astype(jnp.float32)
    dot = lambda a, b: jax.lax.dot_general(a, b, (((1,), (0,)), ((), ())),
                                           preferred_element_type=jnp.float32,
                                           precision=jax.lax.Precision.HIGHEST)
    rows = dot(P, x_ref[...])
    ts = dot(P, s_ref[...])
    ts_ref[...] = ts
    o_ref[...] = rows * jnp.tanh(ts)


def _mlp_kernel(xt_ref, pt_ref, W1x_ref, b1x_ref, W2x_ref, b2x_ref, gx_ref, bx_ref,
                W1s_ref, b1s_ref, W2s_ref, b2s_ref, gs_ref, bs_ref, oa_ref, ob_ref):
    H = W1x_ref.shape[0]

    def rep(row, W1c, b1c, W2t, b2c, gc, bc):
        t1 = jnp.tanh(W1c * row + b1c)
        t2 = jax.lax.dot_general(W2t, t1.astype(jnp.bfloat16),
                                 (((1,), (0,)), ((), ())),
                                 preferred_element_type=jnp.float32) + b2c
        h = jax.nn.relu(t2)
        mu = jnp.mean(h, axis=0, keepdims=True)
        var = jnp.mean((h - mu) ** 2, axis=0, keepdims=True)
        ln = (h - mu) / jnp.sqrt(var + 1e-5) * gc + bc
        m = ln.shape[1] // H
        return ln.astype(jnp.bfloat16).reshape(H, m, H).transpose(1, 2, 0).reshape(m, H * H)

    for l in range(LBLK):
        a = rep(xt_ref[l:l + 1, :], W1x_ref[...], b1x_ref[...], W2x_ref[...],
                b2x_ref[...], gx_ref[...], bx_ref[...])
        oa_ref[pl.ds(l * a.shape[0], a.shape[0]), :] = a
        b = rep(pt_ref[l:l + 1, :], W1s_ref[...], b1s_ref[...], W2s_ref[...],
                b2s_ref[...], gs_ref[...], bs_ref[...])
        ob_ref[pl.ds(l * b.shape[0], b.shape[0]), :] = b


def _reprs(xt, pt, wx, ws):
    L, N = xt.shape
    kk = pt.shape[1]
    H = wx[0].shape[0]
    wcol = pl.BlockSpec((H, 1), lambda i: (0, 0))
    wmat = pl.BlockSpec((H, H), lambda i: (0, 0))
    wspecs = [wcol, wcol, wmat, wcol, wcol, wcol]
    return pl.pallas_call(
        _mlp_kernel,
        grid=(L // LBLK,),
        in_specs=[pl.BlockSpec((LBLK, N), lambda i: (i, 0)),
                  pl.BlockSpec((LBLK, kk), lambda i: (i, 0))] + wspecs + wspecs,
        out_specs=(pl.BlockSpec((LBLK * N // H, N), lambda i: (i, 0)),
                   pl.BlockSpec((LBLK * kk // H, N), lambda i: (i, 0))),
        out_shape=(jax.ShapeDtypeStruct((L * N // H, N), jnp.bfloat16),
                   jax.ShapeDtypeStruct((L * kk // H, N), jnp.bfloat16)),
    )(xt, pt, *wx, *ws)


def _sim_kernel(a_ref, b_ref, o_ref):
    s = jax.lax.dot_general(a_ref[...], b_ref[...], (((1,), (1,)), ((), ())),
                            preferred_element_type=jnp.float32)
    m = jnp.max(s, axis=-1, keepdims=True)
    e = jnp.exp(s - m)
    o_ref[...] = e / jnp.sum(e, axis=-1, keepdims=True)


def kernel(x, edge_index, w_pool, xW1, xb1, xW2, xb2, xg, xbeta, sW1, sb1, sW2, sb2, sg, sbeta):
    N, L = x.shape
    H = xW1.shape[1]
    k = int(np.ceil(0.5 * N))

    score = jnp.matmul(x.astype(jnp.bfloat16), w_pool.astype(jnp.bfloat16),
                       preferred_element_type=jnp.float32) / jnp.linalg.norm(w_pool)
    s2d = score.reshape(N, 1)
    st = score.reshape(1, N)

    rank = pl.pallas_call(
        _rank_kernel,
        grid=(N // RBLK,),
        in_specs=[
            pl.BlockSpec((RBLK, 1), lambda i: (i, 0)),
            pl.BlockSpec((1, N), lambda i: (0, 0)),
        ],
        out_specs=pl.BlockSpec((RBLK, 1), lambda i: (i, 0)),
        out_shape=jax.ShapeDtypeStruct((N, 1), jnp.float32),
    )(s2d, st)

    pooled_x, _ts = pl.pallas_call(
        _pool_kernel,
        grid=(k // PBLK,),
        in_specs=[pl.BlockSpec((1, N), lambda p: (0, 0)),
                  pl.BlockSpec((N, L), lambda p: (0, 0)),
                  pl.BlockSpec((N, 1), lambda p: (0, 0))],
        out_specs=(pl.BlockSpec((PBLK, L), lambda p: (p, 0)),
                   pl.BlockSpec((PBLK, 1), lambda p: (p, 0))),
        out_shape=(jax.ShapeDtypeStruct((k, L), jnp.float32),
                   jax.ShapeDtypeStruct((k, 1), jnp.float32)),
    )(rank.reshape(1, N), x, s2d)

    xt = jnp.transpose(x)
    pt = jnp.transpose(pooled_x)
    prep = lambda W1, b1, W2, b2, g, beta: (
        W1.reshape(H, 1), b1.reshape(H, 1),
        jnp.transpose(W2.astype(jnp.bfloat16)), b2.reshape(H, 1),
        g.reshape(H, 1), beta.reshape(H, 1))
    node_flat, sup_flat = _reprs(xt, pt,
                                 prep(xW1, xb1, xW2, xb2, xg, xbeta),
                                 prep(sW1, sb1, sW2, sb2, sg, sbeta))

    out = pl.pallas_call(
        _sim_kernel,
        grid=(N // BN,),
        in_specs=[
            pl.BlockSpec((BN, L * H), lambda i: (i, 0)),
            pl.BlockSpec((k, L * H), lambda i: (0, 0)),
        ],
        out_specs=pl.BlockSpec((BN, k), lambda i: (i, 0)),
        out_shape=jax.ShapeDtypeStruct((N, k), jnp.float32),
    )(node_flat, sup_flat)
    return out


# speedup vs baseline: 1.0690x; 1.0165x over previous
import jax
import jax.numpy as jnp
import numpy as np
from jax.experimental import pallas as pl

BN = 512
RBLK = 512
PBLK = 512
LBLK = 8


def _rank_kernel(s_ref, st_ref, rank_ref):
    i = pl.program_id(0)
    s_i = s_ref[...]
    s_j = st_ref[...]
    gt = (s_j > s_i).astype(jnp.float32)
    idx_i = i * RBLK + jax.lax.broadcasted_iota(jnp.int32, (RBLK, 1), 0)
    idx_j = jax.lax.broadcasted_iota(jnp.int32, (RBLK, s_j.shape[1]), 1)
    tie = ((s_j == s_i) & (idx_j < idx_i)).astype(jnp.float32)
    rank_ref[...] = jnp.sum(gt + tie, axis=1, keepdims=True)


def _pool_kernel(rt_ref, x_ref, s_ref, o_ref, ts_ref):
    p = pl.program_id(0)
    ranks = rt_ref[...]
    p_col = (p * PBLK + jax.lax.broadcasted_iota(jnp.int32, (PBLK, 1), 0)).astype(jnp.float32)
    P = (ranks == p_col).astype(jnp.float32)
    dot = lambda a, b: jax.lax.dot_general(a, b, (((1,), (0,)), ((), ())),
                                           preferred_element_type=jnp.float32,
                                           precision=jax.lax.Precision.HIGHEST)
    rows = dot(P, x_ref[...])
    ts = dot(P, s_ref[...])
    ts_ref[...] = ts
    o_ref[...] = rows * jnp.tanh(ts)


def _mlp_kernel(xt_ref, pt_ref, W1x_ref, b1x_ref, W2x_ref, b2x_ref, gx_ref, bx_ref,
                W1s_ref, b1s_ref, W2s_ref, b2s_ref, gs_ref, bs_ref, oa_ref, ob_ref):
    H = W1x_ref.shape[0]
    eye = (jax.lax.broadcasted_iota(jnp.int32, (H, H), 0) ==
           jax.lax.broadcasted_iota(jnp.int32, (H, H), 1)).astype(jnp.bfloat16)

    def rep(row, W1c, b1c, W2t, b2c, gc, bc):
        t1 = jnp.tanh(W1c * row + b1c)
        t2 = jax.lax.dot_general(W2t, t1.astype(jnp.bfloat16),
                                 (((1,), (0,)), ((), ())),
                                 preferred_element_type=jnp.float32) + b2c
        h = jax.nn.relu(t2)
        mu = jnp.mean(h, axis=0, keepdims=True)
        var = jnp.mean((h - mu) ** 2, axis=0, keepdims=True)
        ln = (h - mu) / jnp.sqrt(var + 1e-5) * gc + bc
        tr = jax.lax.dot_general(ln.astype(jnp.bfloat16), eye,
                                 (((0,), (0,)), ((), ())),
                                 preferred_element_type=jnp.float32)
        return tr.astype(jnp.bfloat16)

    for l in range(LBLK):
        a = rep(xt_ref[l:l + 1, :], W1x_ref[...], b1x_ref[...], W2x_ref[...],
                b2x_ref[...], gx_ref[...], bx_ref[...])
        oa_ref[pl.ds(l * a.shape[0], a.shape[0]), :] = a
        b = rep(pt_ref[l:l + 1, :], W1s_ref[...], b1s_ref[...], W2s_ref[...],
                b2s_ref[...], gs_ref[...], bs_ref[...])
        ob_ref[pl.ds(l * b.shape[0], b.shape[0]), :] = b


def _reprs(xt, pt, wx, ws):
    L, N = xt.shape
    kk = pt.shape[1]
    H = wx[0].shape[0]
    wcol = pl.BlockSpec((H, 1), lambda i: (0, 0))
    wmat = pl.BlockSpec((H, H), lambda i: (0, 0))
    wspecs = [wcol, wcol, wmat, wcol, wcol, wcol]
    return pl.pallas_call(
        _mlp_kernel,
        grid=(L // LBLK,),
        in_specs=[pl.BlockSpec((LBLK, N), lambda i: (i, 0)),
                  pl.BlockSpec((LBLK, kk), lambda i: (i, 0))] + wspecs + wspecs,
        out_specs=(pl.BlockSpec((LBLK * N, H), lambda i: (i, 0)),
                   pl.BlockSpec((LBLK * kk, H), lambda i: (i, 0))),
        out_shape=(jax.ShapeDtypeStruct((L * N, H), jnp.bfloat16),
                   jax.ShapeDtypeStruct((L * kk, H), jnp.bfloat16)),
    )(xt, pt, *wx, *ws)


def _sim_kernel(a_ref, b_ref, o_ref):
    s = jax.lax.dot_general(a_ref[...], b_ref[...], (((1,), (1,)), ((), ())),
                            preferred_element_type=jnp.float32)
    m = jnp.max(s, axis=-1, keepdims=True)
    e = jnp.exp(s - m)
    o_ref[...] = e / jnp.sum(e, axis=-1, keepdims=True)


def kernel(x, edge_index, w_pool, xW1, xb1, xW2, xb2, xg, xbeta, sW1, sb1, sW2, sb2, sg, sbeta):
    N, L = x.shape
    H = xW1.shape[1]
    k = int(np.ceil(0.5 * N))

    score = jnp.matmul(x.astype(jnp.bfloat16), w_pool.astype(jnp.bfloat16),
                       preferred_element_type=jnp.float32) / jnp.linalg.norm(w_pool)
    s2d = score.reshape(N, 1)
    st = score.reshape(1, N)

    rank = pl.pallas_call(
        _rank_kernel,
        grid=(N // RBLK,),
        in_specs=[
            pl.BlockSpec((RBLK, 1), lambda i: (i, 0)),
            pl.BlockSpec((1, N), lambda i: (0, 0)),
        ],
        out_specs=pl.BlockSpec((RBLK, 1), lambda i: (i, 0)),
        out_shape=jax.ShapeDtypeStruct((N, 1), jnp.float32),
    )(s2d, st)

    pooled_x, _ts = pl.pallas_call(
        _pool_kernel,
        grid=(k // PBLK,),
        in_specs=[pl.BlockSpec((1, N), lambda p: (0, 0)),
                  pl.BlockSpec((N, L), lambda p: (0, 0)),
                  pl.BlockSpec((N, 1), lambda p: (0, 0))],
        out_specs=(pl.BlockSpec((PBLK, L), lambda p: (p, 0)),
                   pl.BlockSpec((PBLK, 1), lambda p: (p, 0))),
        out_shape=(jax.ShapeDtypeStruct((k, L), jnp.float32),
                   jax.ShapeDtypeStruct((k, 1), jnp.float32)),
    )(rank.reshape(1, N), x, s2d)

    xt = jnp.transpose(x)
    pt = jnp.transpose(pooled_x)
    prep = lambda W1, b1, W2, b2, g, beta: (
        W1.reshape(H, 1), b1.reshape(H, 1),
        jnp.transpose(W2.astype(jnp.bfloat16)), b2.reshape(H, 1),
        g.reshape(H, 1), beta.reshape(H, 1))
    a2, b2 = _reprs(xt, pt,
                    prep(xW1, xb1, xW2, xb2, xg, xbeta),
                    prep(sW1, sb1, sW2, sb2, sg, sbeta))
    node_flat = a2.reshape(N, L * H)
    sup_flat = b2.reshape(k, L * H)

    out = pl.pallas_call(
        _sim_kernel,
        grid=(N // BN,),
        in_specs=[
            pl.BlockSpec((BN, L * H), lambda i: (i, 0)),
            pl.BlockSpec((k, L * H), lambda i: (0, 0)),
        ],
        out_specs=pl.BlockSpec((BN, k), lambda i: (i, 0)),
        out_shape=jax.ShapeDtypeStruct((N, k), jnp.float32),
    )(node_flat, sup_flat)
    return out
